# baseline (speedup 1.0000x reference)
import functools

import jax
import jax.numpy as jnp
from jax import lax
from jax.experimental import pallas as pl
from jax.experimental.pallas import tpu as pltpu
from jax.experimental.pallas import tpu_sc as plsc

B, N, C = 8, 16384, 80
BN = 8192

_INFO = plsc.get_sparse_core_info()
_NC, _NS = _INFO.num_cores, _INFO.num_subcores
_NW = _NC * _NS
_PER_W = (B * N) // _NW
_K = 512
_NCHUNK = _PER_W // _K


def _sc_sw_body(oh_ref, wsp_ref, out_ref, buf0, buf1, wv, acc, sem0, sem1):
    wid = lax.axis_index("s") * _NC + lax.axis_index("c")
    base = wid * _PER_W
    row0 = (wid // (N // _PER_W)) * C

    pltpu.sync_copy(wsp_ref, wv)

    bufs = (buf0, buf1)
    sems = (sem0, sem1)

    def copy(i):
        col = (base % N) + i * _K
        return pltpu.make_async_copy(
            oh_ref.at[pl.ds(row0, C), pl.ds(col, _K)], bufs[i % 2], sems[i % 2])

    copy(0).start()
    for i in range(_NCHUNK):
        if i + 1 < _NCHUNK:
            copy(i + 1).start()
        copy(i).wait()
        buf = bufs[i % 2]

        def jbody(j, _):
            a = jnp.zeros((16,), jnp.float32)
            for c in range(C):
                a += wv[pl.ds(c * 16, 16)] * buf[c, pl.ds(j * 16, 16)]
            acc[pl.ds(i * _K + j * 16, 16)] = a
            return 0

        lax.fori_loop(0, _K // 16, jbody, 0)

    pltpu.sync_copy(acc, out_ref.at[pl.ds(base, _PER_W)])


def _sc_sw(oh2d, wsp):
    mesh = plsc.VectorSubcoreMesh(core_axis_name="c", subcore_axis_name="s")
    kern = functools.partial(
        pl.kernel,
        mesh=mesh,
        out_type=jax.ShapeDtypeStruct((B * N,), jnp.float32),
        scratch_types=[
            pltpu.VMEM((C, _K), jnp.float32),
            pltpu.VMEM((C, _K), jnp.float32),
            pltpu.VMEM((C * 16,), jnp.float32),
            pltpu.VMEM((_PER_W,), jnp.float32),
            pltpu.SemaphoreType.DMA,
            pltpu.SemaphoreType.DMA,
        ],
    )(_sc_sw_body)
    return kern(oh2d, wsp)


def _tc_body(x_ref, z_ref, m_ref, sw_ref, o_ref):
    x = x_ref[0]
    z = z_ref[0]
    m = m_ref[0]
    sw = sw_ref[0]

    e = jnp.exp(-jnp.abs(x))
    t = 1.0 + e
    l1p = jnp.log(t)
    r = 1.0 / t
    sig = jnp.where(x >= 0.0, r, 1.0 - r)
    sp = jnp.maximum(x, 0.0) + l1p
    neg = sp * sig * sig
    d = z - sig
    pos = (sp - x * z) * d * d
    o_ref[0] = sw * jnp.where(m != 0, pos, neg)


def kernel(pred_score, gt_score, gt_target_pos_mask, labels_one_hot, weights):
    xT = jnp.transpose(pred_score, (0, 2, 1))
    zT = jnp.transpose(gt_score, (0, 2, 1))
    mT = jnp.transpose(gt_target_pos_mask.view(jnp.int8), (0, 2, 1))
    ohT = jnp.transpose(labels_one_hot, (0, 2, 1))
    oh2d = ohT.reshape(B * C, N)
    wsp = jnp.tile(weights[:, None], (1, 16)).reshape(-1)

    sw = _sc_sw(oh2d, wsp).reshape(B, 1, N)

    grid = (B, N // BN)
    blk = pl.BlockSpec((1, C, BN), lambda b, i: (b, 0, i))
    outT = pl.pallas_call(
        _tc_body,
        grid=grid,
        in_specs=[
            blk,
            blk,
            blk,
            pl.BlockSpec((1, 1, BN), lambda b, i: (b, 0, i)),
        ],
        out_specs=blk,
        out_shape=jax.ShapeDtypeStruct((B, C, N), jnp.float32),
        compiler_params=pltpu.CompilerParams(
            dimension_semantics=("parallel", "parallel"),
            allow_input_fusion=[False, False, True, False],
        ),
    )(xT, zT, mT, sw)
    return jnp.transpose(outT, (0, 2, 1))


# speedup vs baseline: 2.0910x; 2.0910x over previous
import jax
import jax.numpy as jnp
from jax import lax
from jax.experimental import pallas as pl
from jax.experimental.pallas import tpu as pltpu

B, N, C = 8, 16384, 80
BN = 8192


def _body(wm_ref, x_ref, z_ref, m_ref, oh_ref, o_ref):
    x = x_ref[0]
    z = z_ref[0]
    m = m_ref[0]
    oh = oh_ref[0]
    wm = wm_ref[...]

    swf = lax.dot_general(wm, oh, (((0,), (0,)), ((), ())),
                          preferred_element_type=jnp.float32)
    sw = swf[0:C, :]

    e = jnp.exp(-jnp.abs(x))
    t = 1.0 + e
    l1p = jnp.log(t)
    r = 1.0 / t
    sig = jnp.where(x >= 0.0, r, 1.0 - r)
    sp = jnp.maximum(x, 0.0) + l1p
    neg = sp * sig * sig
    d = z - sig
    pos = (sp - x * z) * d * d
    o_ref[0] = sw * jnp.where(m != 0, pos, neg)


def kernel(pred_score, gt_score, gt_target_pos_mask, labels_one_hot, weights):
    xT = jnp.transpose(pred_score, (0, 2, 1))
    zT = jnp.transpose(gt_score, (0, 2, 1))
    mT = jnp.transpose(gt_target_pos_mask.view(jnp.int8), (0, 2, 1))
    ohT = jnp.transpose(labels_one_hot, (0, 2, 1))
    wm = jnp.tile(weights[:, None], (1, 128))

    grid = (B, N // BN)
    blk = pl.BlockSpec((1, C, BN), lambda b, i: (b, 0, i))
    outT = pl.pallas_call(
        _body,
        grid=grid,
        in_specs=[
            pl.BlockSpec((C, 128), lambda b, i: (0, 0)),
            blk,
            blk,
            blk,
            blk,
        ],
        out_specs=blk,
        out_shape=jax.ShapeDtypeStruct((B, C, N), jnp.float32),
        compiler_params=pltpu.CompilerParams(
            dimension_semantics=("parallel", "parallel"),
            allow_input_fusion=[False, False, False, True, False],
        ),
    )(wm, xT, zT, mT, ohT)
    return jnp.transpose(outT, (0, 2, 1))
